# baseline (speedup 1.0000x reference)
import functools

import jax
import jax.numpy as jnp
from jax import lax
from jax.experimental import pallas as pl
from jax.experimental.pallas import tpu as pltpu
from jax.experimental.pallas import tpu_sc as plsc

N = 10000
H = 256
HH = H // 2
NC = 2
NS = 16
CHUNK = 64
IDXB = 32
NB = 4
NP = 10240
ZR = NP // NS

_mesh = plsc.VectorSubcoreMesh(core_axis_name="c", subcore_axis_name="s")


def _zero_vmem_rows(buf, nrows):
    zeros16 = jnp.zeros((16,), jnp.float32)

    def body(i, _):
        for j in range(HH // 16):
            buf[i, pl.ds(j * 16, 16)] = zeros16
        return 0

    lax.fori_loop(0, nrows, body, 0)


def _sc_deg_body(dstp, deg_out, dst_i, ones_v, zeros_v, deg_sh):
    cid = lax.axis_index("c")
    sid = lax.axis_index("s")
    wid = sid * NC + cid
    n_chunks = dstp.shape[0] // (NC * NS * CHUNK)

    ones16 = jnp.ones((16,), jnp.float32)
    zero16 = jnp.zeros((16,), jnp.float32)
    for j in range(CHUNK // 16):
        ones_v[pl.ds(j * 16, 16)] = ones16
        zeros_v[pl.ds(j * 16, 16)] = zero16
    for k in range(ZR // CHUNK):
        pltpu.sync_copy(zeros_v, deg_sh.at[pl.ds(sid * ZR + k * CHUNK, CHUNK)])
    plsc.subcore_barrier()

    def body(g, _):
        base = wid * n_chunks * CHUNK + g * CHUNK
        pltpu.sync_copy(dstp.at[pl.ds(base, CHUNK)], dst_i.at[0])
        pltpu.sync_copy(ones_v, deg_sh.at[dst_i.at[0]], add=True)
        return 0

    lax.fori_loop(0, n_chunks, body, 0)
    plsc.subcore_barrier()
    pltpu.sync_copy(deg_sh.at[pl.ds(sid * ZR, ZR)],
                    deg_out.at[cid, pl.ds(sid * ZR, ZR)])


_sc_deg = functools.partial(
    pl.kernel,
    out_type=jax.ShapeDtypeStruct((NC, NP), jnp.float32),
    mesh=_mesh,
    scratch_types=[
        pltpu.VMEM((1, CHUNK), jnp.int32),
        pltpu.VMEM((CHUNK,), jnp.float32),
        pltpu.VMEM((CHUNK,), jnp.float32),
        pltpu.VMEM_SHARED((NP,), jnp.float32),
    ],
)(_sc_deg_body)


def _sc_scatter_body(table, srcp, dstp, s_out, src_i, dst_i, rows_v, sem_g,
                     sem_s, s_sh):
    cid = lax.axis_index("c")
    sid = lax.axis_index("s")
    n_blocks = srcp.shape[1] // IDXB

    _zero_vmem_rows(rows_v.at[0], CHUNK)
    for k in range(ZR // CHUNK):
        pltpu.sync_copy(rows_v.at[0], s_sh.at[pl.ds(sid * ZR + k * CHUNK, CHUNK)])
    plsc.subcore_barrier()

    def fire_gather(j, b):
        pltpu.async_copy(table.at[cid].at[src_i.at[j]], rows_v.at[b], sem_g)

    def wait_one(sem):
        pltpu.make_async_copy(table.at[cid, pl.ds(0, CHUNK)], rows_v.at[0],
                              sem).wait()

    def blk_body(blk, _):
        pltpu.sync_copy(srcp.at[sid, pl.ds(blk * IDXB, IDXB)], src_i)
        pltpu.sync_copy(dstp.at[sid, pl.ds(blk * IDXB, IDXB)], dst_i)
        fire_gather(0, 0)
        fire_gather(1, 1)

        def body(j, _):
            @pl.when(j >= 2)
            def _():
                wait_one(sem_s)

            @pl.when(j + 2 < IDXB)
            def _():
                fire_gather(j + 2, lax.rem(j + 2, NB))

            wait_one(sem_g)
            pltpu.async_copy(rows_v.at[lax.rem(j, NB)],
                             s_sh.at[dst_i.at[j]], sem_s, add=True)
            return 0

        lax.fori_loop(0, IDXB, body, 0)
        wait_one(sem_s)
        wait_one(sem_s)
        return 0

    lax.fori_loop(0, n_blocks, blk_body, 0)
    plsc.subcore_barrier()
    pltpu.sync_copy(s_sh.at[pl.ds(sid * ZR, ZR)],
                    s_out.at[cid, pl.ds(sid * ZR, ZR)])


def _make_sc_scatter(n_chunks):
    return functools.partial(
        pl.kernel,
        out_type=jax.ShapeDtypeStruct((NC, NP, HH), jnp.float32),
        mesh=_mesh,
        scratch_types=[
            pltpu.VMEM((IDXB, CHUNK), jnp.int32),
            pltpu.VMEM((IDXB, CHUNK), jnp.int32),
            pltpu.VMEM((NB, CHUNK, HH), jnp.float32),
            pltpu.SemaphoreType.DMA,
            pltpu.SemaphoreType.DMA,
            pltpu.VMEM_SHARED((NP, HH), jnp.float32),
        ],
    )(_sc_scatter_body)


def _tc_stage1_body(h0_ref, w_ref, dis_ref, out_ref):
    res = jnp.dot(h0_ref[...], w_ref[...],
                  preferred_element_type=jnp.float32) * dis_ref[...]
    out_ref[0] = res[:, :HH]
    out_ref[1] = res[:, HH:]


def _tc_stage_body(s_ref, hwsp_ref, dis_ref, b_ref, w_ref, out_ref):
    dis = dis_ref[...]
    agg = jnp.concatenate(
        [s_ref[0, :N] + hwsp_ref[0], s_ref[1, :N] + hwsp_ref[1]], axis=1)
    h = jax.nn.relu(agg * dis + b_ref[...])
    res = jnp.dot(h, w_ref[...], preferred_element_type=jnp.float32) * dis
    out_ref[0] = res[:, :HH]
    out_ref[1] = res[:, HH:]


def _tc_final_body(s_ref, hwsp_ref, dis_ref, b_ref, batch_ref, wlin_ref,
                   blin_ref, out_ref):
    agg = jnp.concatenate(
        [s_ref[0, :N] + hwsp_ref[0], s_ref[1, :N] + hwsp_ref[1]], axis=1)
    h = jax.nn.relu(agg * dis_ref[...] + b_ref[...])
    g_iota = lax.broadcasted_iota(jnp.int32, (64, N), 0)
    onehot = jnp.where(batch_ref[...] == g_iota, 1.0, 0.0).astype(jnp.float32)
    sums = jnp.dot(onehot, h, preferred_element_type=jnp.float32)
    cnt = jnp.sum(onehot, axis=1, keepdims=True)
    pooled = sums / jnp.maximum(cnt, 1.0)
    out_ref[...] = jnp.dot(pooled, wlin_ref[...],
                           preferred_element_type=jnp.float32) + blin_ref[...]


def kernel(x, pos, edge_index, batch, W1, b1, W2, b2, W3, b3, Wlin, blin):
    h0 = jnp.concatenate([pos, x], axis=1)
    e = edge_index.shape[1]
    n_chunks = IDXB * (-(-e // (NS * IDXB * CHUNK)))
    pad = NS * n_chunks * CHUNK - e
    srcp = jnp.concatenate([edge_index[0], jnp.zeros((pad,), jnp.int32)])
    dstp = jnp.concatenate([edge_index[1], jnp.full((pad,), N, jnp.int32)])

    srcp3 = srcp.reshape(NS, n_chunks, CHUNK)
    dstp3 = dstp.reshape(NS, n_chunks, CHUNK)
    sc_scatter = _make_sc_scatter(n_chunks)

    degp = _sc_deg(dstp)
    deg = degp[0, :N] + degp[1, :N] + 1.0
    dis = lax.rsqrt(deg).reshape(N, 1)

    tc1 = pl.pallas_call(
        _tc_stage1_body,
        out_shape=jax.ShapeDtypeStruct((NC, N, HH), jnp.float32),
    )
    tcs = pl.pallas_call(
        _tc_stage_body,
        out_shape=jax.ShapeDtypeStruct((NC, N, HH), jnp.float32),
    )
    tcf = pl.pallas_call(
        _tc_final_body,
        out_shape=jax.ShapeDtypeStruct((64, Wlin.shape[1]), jnp.float32),
    )

    hws1 = tc1(h0, W1, dis)
    s1 = sc_scatter(hws1, srcp3, dstp3)
    hws2 = tcs(s1, hws1, dis, b1.reshape(1, H), W2)
    s2 = sc_scatter(hws2, srcp3, dstp3)
    hws3 = tcs(s2, hws2, dis, b2.reshape(1, H), W3)
    s3 = sc_scatter(hws3, srcp3, dstp3)
    return tcf(s3, hws3, dis, b3.reshape(1, H), batch.reshape(1, N),
               Wlin, blin.reshape(1, Wlin.shape[1]))


# speedup vs baseline: 1.0591x; 1.0591x over previous
import functools

import jax
import jax.numpy as jnp
from jax import lax
from jax.experimental import pallas as pl
from jax.experimental.pallas import tpu as pltpu
from jax.experimental.pallas import tpu_sc as plsc

N = 10000
H = 256
HH = H // 2
NC = 2
NS = 16
CHUNK = 128
IDXB = 32
NP = 10240
ZR = NP // NS

_mesh = plsc.VectorSubcoreMesh(core_axis_name="c", subcore_axis_name="s")


def _zero_vmem_rows(buf, nrows):
    zeros16 = jnp.zeros((16,), jnp.float32)

    def body(i, _):
        for j in range(HH // 16):
            buf[i, pl.ds(j * 16, 16)] = zeros16
        return 0

    lax.fori_loop(0, nrows, body, 0)


def _sc_deg_body(dstp, deg_out, dst_i, ones_v, zeros_v, deg_sh):
    cid = lax.axis_index("c")
    sid = lax.axis_index("s")
    wid = sid * NC + cid
    n_chunks = dstp.shape[0] // (NC * NS * CHUNK)

    ones16 = jnp.ones((16,), jnp.float32)
    zero16 = jnp.zeros((16,), jnp.float32)
    for j in range(CHUNK // 16):
        ones_v[pl.ds(j * 16, 16)] = ones16
        zeros_v[pl.ds(j * 16, 16)] = zero16
    for k in range(ZR // CHUNK):
        pltpu.sync_copy(zeros_v, deg_sh.at[pl.ds(sid * ZR + k * CHUNK, CHUNK)])
    plsc.subcore_barrier()

    def body(g, _):
        base = wid * n_chunks * CHUNK + g * CHUNK
        pltpu.sync_copy(dstp.at[pl.ds(base, CHUNK)], dst_i.at[0])
        pltpu.sync_copy(ones_v, deg_sh.at[dst_i.at[0]], add=True)
        return 0

    lax.fori_loop(0, n_chunks, body, 0)
    plsc.subcore_barrier()
    pltpu.sync_copy(deg_sh.at[pl.ds(sid * ZR, ZR)],
                    deg_out.at[cid, pl.ds(sid * ZR, ZR)])


_sc_deg = functools.partial(
    pl.kernel,
    out_type=jax.ShapeDtypeStruct((NC, NP), jnp.float32),
    mesh=_mesh,
    scratch_types=[
        pltpu.VMEM((1, CHUNK), jnp.int32),
        pltpu.VMEM((CHUNK,), jnp.float32),
        pltpu.VMEM((CHUNK,), jnp.float32),
        pltpu.VMEM_SHARED((NP,), jnp.float32),
    ],
)(_sc_deg_body)


def _sc_scatter_body(table, srcp, dstp, s_out, src_i, dst_i, rows_v, sem_g,
                     sem_s, s_sh):
    cid = lax.axis_index("c")
    sid = lax.axis_index("s")
    n_blocks = srcp.shape[1] // IDXB

    _zero_vmem_rows(rows_v.at[0], CHUNK)
    for k in range(ZR // CHUNK):
        pltpu.sync_copy(rows_v.at[0], s_sh.at[pl.ds(sid * ZR + k * CHUNK, CHUNK)])
    plsc.subcore_barrier()

    def drain_one_scatter():
        pltpu.make_async_copy(table.at[cid, pl.ds(0, CHUNK)], rows_v.at[0],
                              sem_s).wait()

    def blk_body(blk, _):
        pltpu.sync_copy(srcp.at[sid, pl.ds(blk * IDXB, IDXB)], src_i)
        pltpu.sync_copy(dstp.at[sid, pl.ds(blk * IDXB, IDXB)], dst_i)

        def body(j, _):
            g = blk * IDXB + j
            b = lax.rem(g, 2)

            @pl.when(g >= 2)
            def _():
                drain_one_scatter()

            pltpu.async_copy(table.at[cid].at[src_i.at[j]], rows_v.at[b],
                             sem_g).wait()
            pltpu.async_copy(rows_v.at[b], s_sh.at[dst_i.at[j]], sem_s,
                             add=True)
            return 0

        lax.fori_loop(0, IDXB, body, 0)
        return 0

    lax.fori_loop(0, n_blocks, blk_body, 0)
    drain_one_scatter()
    drain_one_scatter()
    plsc.subcore_barrier()
    pltpu.sync_copy(s_sh.at[pl.ds(sid * ZR, ZR)],
                    s_out.at[cid, pl.ds(sid * ZR, ZR)])


def _make_sc_scatter(n_chunks):
    return functools.partial(
        pl.kernel,
        out_type=jax.ShapeDtypeStruct((NC, NP, HH), jnp.float32),
        mesh=_mesh,
        scratch_types=[
            pltpu.VMEM((IDXB, CHUNK), jnp.int32),
            pltpu.VMEM((IDXB, CHUNK), jnp.int32),
            pltpu.VMEM((2, CHUNK, HH), jnp.float32),
            pltpu.SemaphoreType.DMA,
            pltpu.SemaphoreType.DMA,
            pltpu.VMEM_SHARED((NP, HH), jnp.float32),
        ],
    )(_sc_scatter_body)


def _tc_stage1_body(h0_ref, w_ref, dis_ref, out_ref):
    res = jnp.dot(h0_ref[...], w_ref[...],
                  preferred_element_type=jnp.float32) * dis_ref[...]
    out_ref[0] = res[:, :HH]
    out_ref[1] = res[:, HH:]


def _tc_stage_body(s_ref, hwsp_ref, dis_ref, b_ref, w_ref, out_ref):
    dis = dis_ref[...]
    agg = jnp.concatenate(
        [s_ref[0, :N] + hwsp_ref[0], s_ref[1, :N] + hwsp_ref[1]], axis=1)
    h = jax.nn.relu(agg * dis + b_ref[...])
    res = jnp.dot(h, w_ref[...], preferred_element_type=jnp.float32) * dis
    out_ref[0] = res[:, :HH]
    out_ref[1] = res[:, HH:]


def _tc_final_body(s_ref, hwsp_ref, dis_ref, b_ref, batch_ref, wlin_ref,
                   blin_ref, out_ref):
    agg = jnp.concatenate(
        [s_ref[0, :N] + hwsp_ref[0], s_ref[1, :N] + hwsp_ref[1]], axis=1)
    h = jax.nn.relu(agg * dis_ref[...] + b_ref[...])
    g_iota = lax.broadcasted_iota(jnp.int32, (64, N), 0)
    onehot = jnp.where(batch_ref[...] == g_iota, 1.0, 0.0).astype(jnp.float32)
    sums = jnp.dot(onehot, h, preferred_element_type=jnp.float32)
    cnt = jnp.sum(onehot, axis=1, keepdims=True)
    pooled = sums / jnp.maximum(cnt, 1.0)
    out_ref[...] = jnp.dot(pooled, wlin_ref[...],
                           preferred_element_type=jnp.float32) + blin_ref[...]


def kernel(x, pos, edge_index, batch, W1, b1, W2, b2, W3, b3, Wlin, blin):
    h0 = jnp.concatenate([pos, x], axis=1)
    e = edge_index.shape[1]
    n_chunks = IDXB * (-(-e // (NS * IDXB * CHUNK)))
    pad = NS * n_chunks * CHUNK - e
    srcp = jnp.concatenate([edge_index[0], jnp.zeros((pad,), jnp.int32)])
    dstp = jnp.concatenate([edge_index[1], jnp.full((pad,), N, jnp.int32)])

    srcp3 = srcp.reshape(NS, n_chunks, CHUNK)
    dstp3 = dstp.reshape(NS, n_chunks, CHUNK)
    sc_scatter = _make_sc_scatter(n_chunks)

    degp = _sc_deg(dstp)
    deg = degp[0, :N] + degp[1, :N] + 1.0
    dis = lax.rsqrt(deg).reshape(N, 1)

    tc1 = pl.pallas_call(
        _tc_stage1_body,
        out_shape=jax.ShapeDtypeStruct((NC, N, HH), jnp.float32),
    )
    tcs = pl.pallas_call(
        _tc_stage_body,
        out_shape=jax.ShapeDtypeStruct((NC, N, HH), jnp.float32),
    )
    tcf = pl.pallas_call(
        _tc_final_body,
        out_shape=jax.ShapeDtypeStruct((64, Wlin.shape[1]), jnp.float32),
    )

    hws1 = tc1(h0, W1, dis)
    s1 = sc_scatter(hws1, srcp3, dstp3)
    hws2 = tcs(s1, hws1, dis, b1.reshape(1, H), W2)
    s2 = sc_scatter(hws2, srcp3, dstp3)
    hws3 = tcs(s2, hws2, dis, b2.reshape(1, H), W3)
    s3 = sc_scatter(hws3, srcp3, dstp3)
    return tcf(s3, hws3, dis, b3.reshape(1, H), batch.reshape(1, N),
               Wlin, blin.reshape(1, Wlin.shape[1]))
